# Initial kernel scaffold; baseline (speedup 1.0000x reference)
import jax
import jax.numpy as jnp
from jax.experimental import pallas as pl


def kernel(pos, batch, W, b):
    raise NotImplementedError("write your pallas kernel here")


# speedup vs baseline: 1.4977x; 1.4977x over previous
import functools

import jax
import jax.numpy as jnp
from jax import lax
from jax.experimental import pallas as pl
from jax.experimental.pallas import tpu as pltpu
from jax.experimental.pallas import tpu_sc as plsc

N = 3200000
NSEG = 16
L = 16
NC, NS = 2, 16
NW = NC * NS
ROWS_W = N // NW
FLAT_W = ROWS_W * 3
CHUNK = 60000
NCHUNK = FLAT_W // CHUNK
GROUP = 3 * L
NEGF = float("-inf")


def _sc_partials(pos_flat, batch):
    mesh = plsc.VectorSubcoreMesh(core_axis_name="c", subcore_axis_name="s")

    @functools.partial(
        pl.kernel,
        mesh=mesh,
        out_type=jax.ShapeDtypeStruct((NW, NSEG * 3, L), jnp.float32),
        scratch_types=[
            pltpu.VMEM((CHUNK,), jnp.float32),
            pltpu.VMEM((NSEG * 3, L), jnp.float32),
            pltpu.VMEM((L,), jnp.int32),
            pltpu.VMEM((L,), jnp.int32),
            pltpu.VMEM((2 * L,), jnp.int32),
            pltpu.SemaphoreType.DMA,
        ],
    )
    def seg_max_kernel(pos_hbm, batch_hbm, out_hbm, buf, accv, idxb, valb,
                       endb, sem):
        cid = lax.axis_index("c")
        sid = lax.axis_index("s")
        wid = sid * NC + cid
        r0 = wid * ROWS_W
        r1 = r0 + ROWS_W
        lanes = lax.iota(jnp.int32, L)
        neg = jnp.full((L,), NEGF, jnp.float32)

        targets = lanes + 1
        lo0 = jnp.full((L,), r0, jnp.int32)
        hi0 = jnp.full((L,), r1, jnp.int32)

        def bs_body(_, carry):
            lo, hi = carry
            active = lo < hi
            mid = lo + lax.shift_right_logical(hi - lo, 1)
            idxb[...] = jnp.minimum(mid, N - 1)
            pltpu.async_copy(batch_hbm.at[idxb], valb, sem).wait()
            v = valb[...]
            ge = v >= targets
            lt = v < targets
            hi = jnp.where(active & ge, mid, hi)
            lo = jnp.where(active & lt, mid + 1, lo)
            return lo, hi

        ends, _ = lax.fori_loop(0, 17, bs_body, (lo0, hi0))
        endb[pl.ds(0, L)] = ends
        endb[pl.ds(L, L)] = jnp.full((L,), r1, jnp.int32)

        bnd = [r0]
        for s in range(NSEG):
            bnd.append(endb[pl.ds(s, L)][0])

        for i in range(NSEG * 3):
            accv[i] = neg

        wf0 = r0 * 3

        def chunk_body(c, _):
            chunk_lo = pl.multiple_of(wf0 + c * CHUNK, 8)
            pltpu.sync_copy(pos_hbm.at[pl.ds(chunk_lo, CHUNK)], buf)
            for s in range(NSEG):
                a = jnp.maximum(bnd[s] * 3, chunk_lo)
                e = jnp.minimum(bnd[s + 1] * 3, chunk_lo + CHUNK)

                @pl.when(a < e)
                def _():
                    g0 = (a - chunk_lo) // GROUP
                    g1 = (e - chunk_lo + (GROUP - 1)) // GROUP

                    def g_body(g, acc):
                        base = chunk_lo + g * GROUP
                        off = g * GROUP
                        new = []
                        for p in range(3):
                            v = buf[pl.ds(off + p * L, L)]
                            fl = base + p * L + lanes
                            m = (fl >= a) & (fl < e)
                            new.append(
                                jnp.maximum(acc[p], jnp.where(m, v, neg)))
                        return tuple(new)

                    acc = lax.fori_loop(g0, g1, g_body, (neg, neg, neg))
                    for p in range(3):
                        accv[3 * s + p] = jnp.maximum(accv[3 * s + p], acc[p])
            return 0

        lax.fori_loop(0, NCHUNK, chunk_body, 0)

        pltpu.sync_copy(accv, out_hbm.at[wid])

    return seg_max_kernel(pos_flat, batch)


def _merge_body(x_ref, o_ref):
    x = x_ref[...].reshape(NW, NSEG, 3, L)
    p_idx = lax.broadcasted_iota(jnp.int32, (NW, NSEG, 3, L), 2)
    l_idx = lax.broadcasted_iota(jnp.int32, (NW, NSEG, 3, L), 3)
    pat = (L * p_idx + l_idx) % 3
    cols = []
    for d in range(3):
        cols.append(jnp.max(jnp.where(pat == d, x, NEGF), axis=(0, 2, 3)))
    o_ref[...] = jnp.stack(cols, axis=1)


def _tc_merge(partials):
    return pl.pallas_call(
        _merge_body,
        out_shape=jax.ShapeDtypeStruct((NSEG, 3), jnp.float32),
    )(partials)


def kernel(pos, batch, W, b):
    partials = _sc_partials(pos.reshape(-1), batch)
    x = _tc_merge(partials)
    new_pos = jnp.zeros((NSEG, 6), dtype=pos.dtype)
    new_batch = jnp.arange(NSEG, dtype=jnp.int64)
    return (x, new_pos, new_batch)
